# Initial kernel scaffold; baseline (speedup 1.0000x reference)
import jax
import jax.numpy as jnp
from jax.experimental import pallas as pl


def kernel(q, k, P_prev, log_gain, output_scale):
    raise NotImplementedError("write your pallas kernel here")


# speedup vs baseline: 69.7408x; 69.7408x over previous
import functools

import jax
import jax.numpy as jnp
from jax import lax
from jax.experimental import pallas as pl
from jax.experimental.pallas import tpu as pltpu

_B, _L, _D = 4, 1024, 256
_C = 256
_NC = _L // _C

_PREC = lax.Precision.HIGHEST


def _dot_tt(a, b):
    return lax.dot_general(a, b, (((1,), (1,)), ((), ())),
                           preferred_element_type=jnp.float32,
                           precision=_PREC)


def _body(q_ref, k_ref, pprev_ref, gain_ref, oscale_ref,
          qout_ref, pfin_ref, p_scr):
    c = pl.program_id(1)

    @pl.when(c == 0)
    def _():
        p_scr[...] = pprev_ref[0]

    P = p_scr[...]
    Q = q_ref[0]
    K = k_ref[0]

    row = lax.broadcasted_iota(jnp.int32, (_C, _C), 0)
    col = lax.broadcasted_iota(jnp.int32, (_C, _C), 1)

    num_inter = _dot_tt(Q, P)
    S = _dot_tt(Q, K)
    S_causal = jnp.where(col <= row, S, 0.0)
    num_intra = jnp.dot(S_causal, K,
                        preferred_element_type=jnp.float32,
                        precision=_PREC)
    numer = num_inter + num_intra

    G = _dot_tt(K, K)
    G2 = G * G
    W = jnp.where(col < row, 2.0, jnp.where(col == row, 1.0, 0.0))
    w_row = jnp.sum(G2 * W, axis=1, keepdims=True)

    KP = jnp.dot(K, P, preferred_element_type=jnp.float32,
                 precision=_PREC)
    d = jnp.sum(KP * K, axis=1, keepdims=True)

    tri = jnp.where(col <= row, 1.0, 0.0)
    cs = jnp.dot(tri, 2.0 * d + w_row,
                 preferred_element_type=jnp.float32,
                 precision=_PREC)
    F_in = jnp.sum(P * P)
    fro = jnp.sqrt(F_in + cs)

    q_aligned = numer / (fro + 1e-7)
    gain = jnp.exp(gain_ref[...])
    qout_ref[0] = jnp.tanh(q_aligned * gain) * oscale_ref[...]

    P_new = P + lax.dot_general(K, K, (((0,), (0,)), ((), ())),
                                preferred_element_type=jnp.float32,
                                precision=_PREC)
    p_scr[...] = P_new

    @pl.when(c == _NC - 1)
    def _():
        pfin_ref[0] = P_new


@jax.jit
def kernel(q, k, P_prev, log_gain, output_scale):
    gain2d = log_gain.reshape(1, _D)
    oscale2d = output_scale.reshape(1, _D)

    q_out, P_final = pl.pallas_call(
        _body,
        out_shape=(
            jax.ShapeDtypeStruct((_B, _L, _D), jnp.float32),
            jax.ShapeDtypeStruct((_B, _D, _D), jnp.float32),
        ),
        grid=(_B, _NC),
        in_specs=[
            pl.BlockSpec((1, _C, _D), lambda b, c: (b, c, 0)),
            pl.BlockSpec((1, _C, _D), lambda b, c: (b, c, 0)),
            pl.BlockSpec((1, _D, _D), lambda b, c: (b, 0, 0)),
            pl.BlockSpec((1, _D), lambda b, c: (0, 0)),
            pl.BlockSpec((1, _D), lambda b, c: (0, 0)),
        ],
        out_specs=(
            pl.BlockSpec((1, _C, _D), lambda b, c: (b, c, 0)),
            pl.BlockSpec((1, _D, _D), lambda b, c: (b, 0, 0)),
        ),
        scratch_shapes=[pltpu.VMEM((_D, _D), jnp.float32)],
        compiler_params=pltpu.CompilerParams(
            dimension_semantics=("parallel", "arbitrary"),
        ),
        name="qkproj_chunked",
    )(q, k, P_prev, gain2d, oscale2d)
    return q_out, P_final


# speedup vs baseline: 155.9449x; 2.2361x over previous
import functools

import jax
import jax.numpy as jnp
from jax import lax
from jax.experimental import pallas as pl
from jax.experimental.pallas import tpu as pltpu

_B, _L, _D = 4, 1024, 256
_C = 256
_NC = _L // _C

_PREC = None


def _dot_tt(a, b):
    return lax.dot_general(a, b, (((1,), (1,)), ((), ())),
                           preferred_element_type=jnp.float32,
                           precision=_PREC)


def _body(q_ref, k_ref, pprev_ref, gain_ref, oscale_ref,
          qout_ref, pfin_ref, p_scr):
    c = pl.program_id(1)

    @pl.when(c == 0)
    def _():
        p_scr[...] = pprev_ref[0]

    P = p_scr[...]
    Q = q_ref[0]
    K = k_ref[0]

    row = lax.broadcasted_iota(jnp.int32, (_C, _C), 0)
    col = lax.broadcasted_iota(jnp.int32, (_C, _C), 1)

    num_inter = _dot_tt(Q, P)
    S = _dot_tt(Q, K)
    S_causal = jnp.where(col <= row, S, 0.0)
    num_intra = jnp.dot(S_causal, K,
                        preferred_element_type=jnp.float32,
                        precision=_PREC)
    numer = num_inter + num_intra

    G = _dot_tt(K, K)
    G2 = G * G
    W = jnp.where(col < row, 2.0, jnp.where(col == row, 1.0, 0.0))
    w_row = jnp.sum(G2 * W, axis=1, keepdims=True)

    KP = jnp.dot(K, P, preferred_element_type=jnp.float32,
                 precision=_PREC)
    d = jnp.sum(KP * K, axis=1, keepdims=True)

    tri = jnp.where(col <= row, 1.0, 0.0)
    cs = jnp.dot(tri, 2.0 * d + w_row,
                 preferred_element_type=jnp.float32,
                 precision=_PREC)
    F_in = jnp.sum(P * P)
    fro = jnp.sqrt(F_in + cs)

    q_aligned = numer / (fro + 1e-7)
    gain = jnp.exp(gain_ref[...])
    qout_ref[0] = jnp.tanh(q_aligned * gain) * oscale_ref[...]

    P_new = P + lax.dot_general(K, K, (((0,), (0,)), ((), ())),
                                preferred_element_type=jnp.float32,
                                precision=_PREC)
    p_scr[...] = P_new

    @pl.when(c == _NC - 1)
    def _():
        pfin_ref[0] = P_new


@jax.jit
def kernel(q, k, P_prev, log_gain, output_scale):
    gain2d = log_gain.reshape(1, _D)
    oscale2d = output_scale.reshape(1, _D)

    q_out, P_final = pl.pallas_call(
        _body,
        out_shape=(
            jax.ShapeDtypeStruct((_B, _L, _D), jnp.float32),
            jax.ShapeDtypeStruct((_B, _D, _D), jnp.float32),
        ),
        grid=(_B, _NC),
        in_specs=[
            pl.BlockSpec((1, _C, _D), lambda b, c: (b, c, 0)),
            pl.BlockSpec((1, _C, _D), lambda b, c: (b, c, 0)),
            pl.BlockSpec((1, _D, _D), lambda b, c: (b, 0, 0)),
            pl.BlockSpec((1, _D), lambda b, c: (0, 0)),
            pl.BlockSpec((1, _D), lambda b, c: (0, 0)),
        ],
        out_specs=(
            pl.BlockSpec((1, _C, _D), lambda b, c: (b, c, 0)),
            pl.BlockSpec((1, _D, _D), lambda b, c: (b, 0, 0)),
        ),
        scratch_shapes=[pltpu.VMEM((_D, _D), jnp.float32)],
        compiler_params=pltpu.CompilerParams(
            dimension_semantics=("parallel", "arbitrary"),
        ),
        name="qkproj_chunked",
    )(q, k, P_prev, gain2d, oscale2d)
    return q_out, P_final


# speedup vs baseline: 169.7460x; 1.0885x over previous
import functools

import jax
import jax.numpy as jnp
from jax import lax
from jax.experimental import pallas as pl
from jax.experimental.pallas import tpu as pltpu

_B, _L, _D = 4, 1024, 256
_C = 256
_NC = _L // _C

_PREC = None


def _dot_tt(a, b):
    return lax.dot_general(a, b, (((1,), (1,)), ((), ())),
                           preferred_element_type=jnp.float32,
                           precision=_PREC)


def _body(q_ref, k_ref, pprev_ref, gain_ref, oscale_ref,
          qout_ref, pfin_ref, p_scr):
    c = pl.program_id(1)

    @pl.when(c == 0)
    def _():
        p_scr[...] = pprev_ref[0]

    P = p_scr[...]
    Q = q_ref[0]
    K = k_ref[0]
    QK = jnp.concatenate([Q, K], axis=0)

    row = lax.broadcasted_iota(jnp.int32, (_C, _C), 0)
    col = lax.broadcasted_iota(jnp.int32, (_C, _C), 1)

    A = _dot_tt(QK, P)
    num_inter = A[:_C]
    d = jnp.sum(A[_C:] * K, axis=1, keepdims=True)

    T = _dot_tt(QK, K)
    S = T[:_C]
    G = T[_C:]
    S_causal = jnp.where(col <= row, S, 0.0)
    num_intra = jnp.dot(S_causal, K,
                        preferred_element_type=jnp.float32,
                        precision=_PREC)
    numer = num_inter + num_intra

    G2 = G * G
    W = jnp.where(col < row, 2.0, jnp.where(col == row, 1.0, 0.0))
    w_row = jnp.sum(G2 * W, axis=1, keepdims=True)

    tri = jnp.where(col <= row, 1.0, 0.0)
    cs = jnp.dot(tri, 2.0 * d + w_row,
                 preferred_element_type=jnp.float32,
                 precision=_PREC)
    F_in = jnp.sum(P * P)
    fro = jnp.sqrt(F_in + cs)

    q_aligned = numer / (fro + 1e-7)
    gain = jnp.exp(gain_ref[...])
    qout_ref[0] = jnp.tanh(q_aligned * gain) * oscale_ref[...]

    P_new = P + lax.dot_general(K, K, (((0,), (0,)), ((), ())),
                                preferred_element_type=jnp.float32,
                                precision=_PREC)
    p_scr[...] = P_new

    @pl.when(c == _NC - 1)
    def _():
        pfin_ref[0] = P_new


@jax.jit
def kernel(q, k, P_prev, log_gain, output_scale):
    gain2d = log_gain.reshape(1, _D)
    oscale2d = output_scale.reshape(1, _D)

    q_out, P_final = pl.pallas_call(
        _body,
        out_shape=(
            jax.ShapeDtypeStruct((_B, _L, _D), jnp.float32),
            jax.ShapeDtypeStruct((_B, _D, _D), jnp.float32),
        ),
        grid=(_B, _NC),
        in_specs=[
            pl.BlockSpec((1, _C, _D), lambda b, c: (b, c, 0)),
            pl.BlockSpec((1, _C, _D), lambda b, c: (b, c, 0)),
            pl.BlockSpec((1, _D, _D), lambda b, c: (b, 0, 0)),
            pl.BlockSpec((1, _D), lambda b, c: (0, 0)),
            pl.BlockSpec((1, _D), lambda b, c: (0, 0)),
        ],
        out_specs=(
            pl.BlockSpec((1, _C, _D), lambda b, c: (b, c, 0)),
            pl.BlockSpec((1, _D, _D), lambda b, c: (b, 0, 0)),
        ),
        scratch_shapes=[pltpu.VMEM((_D, _D), jnp.float32)],
        compiler_params=pltpu.CompilerParams(
            dimension_semantics=("parallel", "arbitrary"),
        ),
        name="qkproj_chunked",
    )(q, k, P_prev, gain2d, oscale2d)
    return q_out, P_final
